# Initial kernel scaffold; baseline (speedup 1.0000x reference)
import jax
import jax.numpy as jnp
from jax.experimental import pallas as pl


def kernel(x, table):
    raise NotImplementedError("write your pallas kernel here")


# speedup vs baseline: 7.0240x; 7.0240x over previous
import functools

import jax
import jax.numpy as jnp
from jax import lax
from jax.experimental import pallas as pl
from jax.experimental.pallas import tpu as pltpu
from jax.experimental.pallas import tpu_sc as plsc

_NC = 2
_NS = 16
_NW = _NC * _NS
_CHUNK = 128


def _make_gather(n_rows, d, n_chunks_w):
    mesh = plsc.VectorSubcoreMesh(core_axis_name="c", subcore_axis_name="s")

    @functools.partial(
        pl.kernel,
        mesh=mesh,
        out_type=jax.ShapeDtypeStruct((n_rows, d), jnp.float32),
        scratch_types=[
            pltpu.VMEM((n_chunks_w, _CHUNK), jnp.int32),
            pltpu.VMEM((_CHUNK, d), jnp.float32),
            pltpu.SemaphoreType.DMA,
        ],
    )
    def k(idx_hbm, table_hbm, out_hbm, idx_v, rows_v, sem):
        wid = lax.axis_index("s") * _NC + lax.axis_index("c")
        pltpu.sync_copy(idx_hbm.at[pl.ds(wid * n_chunks_w, n_chunks_w)], idx_v)
        base = wid * (n_chunks_w * _CHUNK)

        def chunk(j, carry):
            pltpu.async_copy(table_hbm.at[idx_v.at[j]], rows_v, sem).wait()
            pltpu.sync_copy(rows_v, out_hbm.at[pl.ds(base + j * _CHUNK, _CHUNK)])
            return carry

        lax.fori_loop(0, n_chunks_w, chunk, 0)

    return k


def kernel(x, table):
    b, s = x.shape
    _, d = table.shape
    n = b * s
    n_chunks_w = n // (_NW * _CHUNK)
    idx = x.reshape(_NW * n_chunks_w, _CHUNK).astype(jnp.int32)
    out = _make_gather(n, d, n_chunks_w)(idx, table)
    return out.reshape(b, s, d)


# speedup vs baseline: 10.3374x; 1.4717x over previous
import functools

import jax
import jax.numpy as jnp
from jax import lax
from jax.experimental import pallas as pl
from jax.experimental.pallas import tpu as pltpu
from jax.experimental.pallas import tpu_sc as plsc

_NC = 2
_NS = 16
_NW = _NC * _NS
_CHUNK = 128
_G = 2
_NSLOT = 2


def _make_gather(n_chunks, d, n_chunks_w):
    mesh = plsc.VectorSubcoreMesh(core_axis_name="c", subcore_axis_name="s")
    n_rounds = n_chunks_w // (_G)

    @functools.partial(
        pl.kernel,
        mesh=mesh,
        out_type=jax.ShapeDtypeStruct((n_chunks, _CHUNK, d), jnp.float32),
        scratch_types=[
            pltpu.VMEM((n_chunks_w, _CHUNK), jnp.int32),
            pltpu.VMEM((_NSLOT * _G, _CHUNK, d), jnp.float32),
            pltpu.SemaphoreType.DMA,
            pltpu.SemaphoreType.DMA,
            pltpu.SemaphoreType.DMA,
        ],
    )
    def k(idx_hbm, table_hbm, out_hbm, idx_v, rows_v, gsem, ssem0, ssem1):
        wid = lax.axis_index("s") * _NC + lax.axis_index("c")
        pltpu.sync_copy(idx_hbm.at[pl.ds(wid * n_chunks_w, n_chunks_w)], idx_v)
        cbase = wid * n_chunks_w
        ssems = (ssem0, ssem1)

        def g_copy(r, s, c):
            return pltpu.make_async_copy(
                table_hbm.at[idx_v.at[r * _G + c]],
                rows_v.at[s * _G + c], gsem)

        def s_copy(r, s):
            return pltpu.make_async_copy(
                rows_v.at[pl.ds(s * _G, _G)],
                out_hbm.at[pl.ds(cbase + r * _G, _G)], ssems[s])

        for s in range(_NSLOT):
            for c in range(_G):
                g_copy(s, s, c).start()

        def body(i, carry):
            for s in range(_NSLOT):
                r = i * _NSLOT + s
                for c in range(_G):
                    g_copy(r, s, c).wait()
                s_copy(r, s).start()
                nr = r + _NSLOT

                @pl.when(nr < n_rounds)
                def _():
                    s_copy(r, s).wait()
                    for c in range(_G):
                        g_copy(nr, s, c).start()
            return carry

        lax.fori_loop(0, n_rounds // _NSLOT, body, 0)
        for s in range(_NSLOT):
            s_copy(n_rounds - _NSLOT + s, s).wait()

    return k


def kernel(x, table):
    b, s = x.shape
    _, d = table.shape
    n = b * s
    n_chunks = n // _CHUNK
    n_chunks_w = n_chunks // _NW
    idx = x.reshape(n_chunks, _CHUNK).astype(jnp.int32)
    out = _make_gather(n_chunks, d, n_chunks_w)(idx, table)
    return out.reshape(b, s, d)
